# Initial kernel scaffold; baseline (speedup 1.0000x reference)
import jax
import jax.numpy as jnp
from jax.experimental import pallas as pl


def kernel(atom_feats, bond_feats, a2b, b2a, b2revb, batch, W_i, W_h, W_o_w, W_o_b):
    raise NotImplementedError("write your pallas kernel here")


# speedup vs baseline: 3.4866x; 3.4866x over previous
import functools

import jax
import jax.numpy as jnp
from jax import lax
from jax.experimental import pallas as pl
from jax.experimental.pallas import tpu as pltpu
from jax.experimental.pallas import tpu_sc as plsc

N_ATOMS = 10000
N_BONDS = 160000
MAX_NB = 6
ATOM_F = 128
BOND_F = 16
HID = 128
DEPTH = 4
NG = 512

_NC = 2
_NS = 16
_NW = _NC * _NS

_E_CH = 128
_E_NCH = 40
E_PAD = _NW * _E_NCH * _E_CH
_A_CH = 80
_A_NCH = 4
N_PAD = _NW * _A_CH * _A_NCH

_MESH = dict(core_axis_name="c", subcore_axis_name="s")


def _wid():
    return lax.axis_index("s") * _NC + lax.axis_index("c")


def _make_dual_gather(w1, w2, dtype2):
    @functools.partial(
        pl.kernel,
        mesh=plsc.VectorSubcoreMesh(**_MESH),
        out_type=[
            jax.ShapeDtypeStruct((E_PAD, w1), jnp.float32),
            jax.ShapeDtypeStruct((E_PAD, w2), dtype2),
        ],
        scratch_types=[
            pltpu.VMEM((_E_CH,), jnp.int32),
            pltpu.VMEM((_E_CH,), jnp.int32),
            pltpu.VMEM((_E_CH, w1), jnp.float32),
            pltpu.VMEM((_E_CH, w2), dtype2),
            pltpu.SemaphoreType.DMA,
            pltpu.SemaphoreType.DMA,
        ],
    )
    def k(t1, i1, t2, i2, o1, o2, iv1, iv2, r1, r2, sm1, sm2):
        wid = _wid()

        def body(g, carry):
            base = wid * (_E_NCH * _E_CH) + g * _E_CH
            pltpu.sync_copy(i1.at[pl.ds(base, _E_CH)], iv1)
            pltpu.sync_copy(i2.at[pl.ds(base, _E_CH)], iv2)
            c1 = pltpu.async_copy(t1.at[iv1], r1, sm1)
            c2 = pltpu.async_copy(t2.at[iv2], r2, sm2)
            c1.wait()
            c2.wait()
            pltpu.sync_copy(r1, o1.at[pl.ds(base, _E_CH)])
            pltpu.sync_copy(r2, o2.at[pl.ds(base, _E_CH)])
            return carry

        lax.fori_loop(0, _E_NCH, body, 0)

    return k


_gab = _make_dual_gather(ATOM_F, 128, jnp.float32)
_g12 = _make_dual_gather(HID, HID, jnp.float32)


@functools.partial(
    pl.kernel,
    mesh=plsc.VectorSubcoreMesh(**_MESH),
    out_type=jax.ShapeDtypeStruct((N_PAD, HID), jnp.float32),
    scratch_types=(
        [pltpu.VMEM((_A_CH,), jnp.int32) for _ in range(MAX_NB)]
        + [pltpu.VMEM((_A_CH, HID), jnp.float32) for _ in range(MAX_NB)]
        + [pltpu.SemaphoreType.DMA for _ in range(MAX_NB)]
    ),
)
def _sk(msg, cols, s_out, i0, i1, i2, i3, i4, i5, r0, r1, r2, r3, r4, r5,
        m0, m1, m2, m3, m4, m5):
    wid = _wid()
    ivs = (i0, i1, i2, i3, i4, i5)
    rs = (r0, r1, r2, r3, r4, r5)
    sems = (m0, m1, m2, m3, m4, m5)

    def body(g, carry):
        base = wid * (_A_NCH * _A_CH) + g * _A_CH
        for j in range(MAX_NB):
            pltpu.sync_copy(cols.at[pl.ds(j * N_PAD + base, _A_CH)], ivs[j])
        copies = [pltpu.async_copy(msg.at[ivs[j]], rs[j], sems[j])
                  for j in range(MAX_NB)]
        for c in copies:
            c.wait()

        def add_row(rr, cc):
            for kk in range(HID // 16):
                sl = pl.ds(kk * 16, 16)
                r0[rr, sl] = (r0[rr, sl] + r1[rr, sl] + r2[rr, sl]
                              + r3[rr, sl] + r4[rr, sl] + r5[rr, sl])
            return cc

        lax.fori_loop(0, _A_CH, add_row, 0)
        pltpu.sync_copy(r0, s_out.at[pl.ds(base, _A_CH)])
        return carry

    lax.fori_loop(0, _A_NCH, body, 0)


_BE = 2048
_EG = E_PAD // _BE


def _proj_body(a_ref, b_ref, g3_ref, br_ref, wa_ref, wb_ref,
               ip_ref, m0_ref, c_ref):
    ip = jnp.dot(a_ref[...], wa_ref[...], preferred_element_type=jnp.float32)
    ip = ip + jnp.dot(b_ref[...], wb_ref[...], preferred_element_type=jnp.float32)
    ip_ref[...] = ip
    m0_ref[...] = jnp.maximum(ip, 0.0)
    eq = (g3_ref[...] == br_ref[...]).astype(jnp.float32)
    c_ref[...] = jnp.sum(eq, axis=1, keepdims=True)


def _proj(A, bond_p, G3, brev_col, W_a, W_b):
    return pl.pallas_call(
        _proj_body,
        grid=(_EG,),
        in_specs=[
            pl.BlockSpec((_BE, ATOM_F), lambda i: (i, 0)),
            pl.BlockSpec((_BE, BOND_F), lambda i: (i, 0)),
            pl.BlockSpec((_BE, 128), lambda i: (i, 0)),
            pl.BlockSpec((_BE, 1), lambda i: (i, 0)),
            pl.BlockSpec((ATOM_F, HID), lambda i: (0, 0)),
            pl.BlockSpec((BOND_F, HID), lambda i: (0, 0)),
        ],
        out_specs=[
            pl.BlockSpec((_BE, HID), lambda i: (i, 0)),
            pl.BlockSpec((_BE, HID), lambda i: (i, 0)),
            pl.BlockSpec((_BE, 1), lambda i: (i, 0)),
        ],
        out_shape=[
            jax.ShapeDtypeStruct((E_PAD, HID), jnp.float32),
            jax.ShapeDtypeStruct((E_PAD, HID), jnp.float32),
            jax.ShapeDtypeStruct((E_PAD, 1), jnp.float32),
        ],
    )(A, bond_p, G3, brev_col, W_a, W_b)


def _depth_body(g1_ref, g2_ref, c_ref, ip_ref, wh_ref, out_ref):
    nei = g1_ref[...] - c_ref[...] * g2_ref[...]
    h = jnp.dot(nei, wh_ref[...], preferred_element_type=jnp.float32)
    out_ref[...] = jnp.maximum(h + ip_ref[...], 0.0)


def _depth(G1, G2, c, ip, W_h2):
    return pl.pallas_call(
        _depth_body,
        grid=(_EG,),
        in_specs=[
            pl.BlockSpec((_BE, HID), lambda i: (i, 0)),
            pl.BlockSpec((_BE, HID), lambda i: (i, 0)),
            pl.BlockSpec((_BE, 1), lambda i: (i, 0)),
            pl.BlockSpec((_BE, HID), lambda i: (i, 0)),
            pl.BlockSpec((HID, HID), lambda i: (0, 0)),
        ],
        out_specs=pl.BlockSpec((_BE, HID), lambda i: (i, 0)),
        out_shape=jax.ShapeDtypeStruct((E_PAD, HID), jnp.float32),
    )(G1, G2, c, ip, W_h2)


_BN = 2000
_NBLK = N_ATOMS // _BN


def _out_body(af_ref, s_ref, bt_ref, woa_ref, woh_ref, bias_ref,
              out_ref, acc_ref, cnt_ref):
    i = pl.program_id(0)

    @pl.when(i == 0)
    def _():
        acc_ref[...] = jnp.zeros_like(acc_ref)
        cnt_ref[...] = jnp.zeros_like(cnt_ref)

    h = jnp.dot(af_ref[...], woa_ref[...], preferred_element_type=jnp.float32)
    h = h + jnp.dot(s_ref[...], woh_ref[...], preferred_element_type=jnp.float32)
    h = jnp.maximum(h + bias_ref[...], 0.0)
    gi = lax.broadcasted_iota(jnp.int32, (NG, _BN), 0)
    bt = bt_ref[...].reshape(1, _BN)
    p = (gi == bt).astype(jnp.float32)
    acc_ref[...] += jnp.dot(p, h, preferred_element_type=jnp.float32)
    cnt_ref[...] += jnp.sum(p, axis=1, keepdims=True)

    @pl.when(i == _NBLK - 1)
    def _():
        out_ref[...] = acc_ref[...] / jnp.maximum(cnt_ref[...], 1.0)


def _outk(atom_feats, s_fin, batch_row, W_oa, W_oh, bias):
    return pl.pallas_call(
        _out_body,
        grid=(_NBLK,),
        in_specs=[
            pl.BlockSpec((_BN, ATOM_F), lambda i: (i, 0)),
            pl.BlockSpec((_BN, HID), lambda i: (i, 0)),
            pl.BlockSpec((1, 1, _BN), lambda i: (i, 0, 0)),
            pl.BlockSpec((ATOM_F, HID), lambda i: (0, 0)),
            pl.BlockSpec((HID, HID), lambda i: (0, 0)),
            pl.BlockSpec((1, HID), lambda i: (0, 0)),
        ],
        out_specs=pl.BlockSpec((NG, HID), lambda i: (0, 0)),
        out_shape=jax.ShapeDtypeStruct((NG, HID), jnp.float32),
        scratch_shapes=[
            pltpu.VMEM((NG, HID), jnp.float32),
            pltpu.VMEM((NG, HID), jnp.float32),
        ],
    )(atom_feats, s_fin, batch_row, W_oa, W_oh, bias)


def kernel(atom_feats, bond_feats, a2b, b2a, b2revb, batch,
           W_i, W_h, W_o_w, W_o_b):
    b2a_p = jnp.pad(b2a, (0, E_PAD - N_BONDS))
    b2revb_p = jnp.pad(b2revb, (0, E_PAD - N_BONDS))
    bond_p = jnp.pad(bond_feats, ((0, E_PAD - N_BONDS), (0, 0)))
    a2b_rows = jnp.pad(a2b, ((0, N_PAD - N_ATOMS), (0, 0)))
    a2b_cols = a2b_rows.T.reshape(-1)
    a2bf = jnp.concatenate(
        [a2b_rows.astype(jnp.float32),
         jnp.full((N_PAD, 128 - MAX_NB), -1.0, jnp.float32)], axis=1)
    brev_col = b2revb_p.astype(jnp.float32).reshape(E_PAD, 1)
    batch_row = batch.reshape(_NBLK, 1, _BN)
    W_a = W_i[:, :ATOM_F].T
    W_b = W_i[:, ATOM_F:].T
    W_h2 = W_h.T
    W_oa = W_o_w[:, :ATOM_F].T
    W_oh = W_o_w[:, ATOM_F:].T
    bias = W_o_b.reshape(1, HID)

    A, G3 = _gab(atom_feats, b2a_p, a2bf, b2a_p)
    ip, msg, c = _proj(A, bond_p, G3, brev_col, W_a, W_b)
    for _ in range(DEPTH - 1):
        s = _sk(msg, a2b_cols)
        G1, G2 = _g12(s, b2a_p, msg, b2revb_p)
        msg = _depth(G1, G2, c, ip, W_h2)
    s = _sk(msg, a2b_cols)
    return _outk(atom_feats, s, batch_row, W_oa, W_oh, bias)


# speedup vs baseline: 3.8546x; 1.1056x over previous
import functools

import jax
import jax.numpy as jnp
from jax import lax
from jax.experimental import pallas as pl
from jax.experimental.pallas import tpu as pltpu
from jax.experimental.pallas import tpu_sc as plsc

N_ATOMS = 10000
N_BONDS = 160000
MAX_NB = 6
ATOM_F = 128
BOND_F = 16
HID = 128
DEPTH = 4
NG = 512

_NC = 2
_NS = 16
_NW = _NC * _NS

_E_CH = 128
_E_NCH = 40
E_PAD = _NW * _E_NCH * _E_CH
_A_CH = 64
_A_NCH = 5
N_PAD = _NW * _A_CH * _A_NCH

_MESH = dict(core_axis_name="c", subcore_axis_name="s")


def _wid():
    return lax.axis_index("s") * _NC + lax.axis_index("c")


@functools.partial(
    pl.kernel,
    mesh=plsc.VectorSubcoreMesh(**_MESH),
    out_type=jax.ShapeDtypeStruct((E_PAD, 2 * ATOM_F), jnp.float32),
    scratch_types=(
        [pltpu.VMEM((_E_CH,), jnp.int32) for _ in range(2)]
        + [pltpu.VMEM((_E_CH, 2 * ATOM_F), jnp.float32) for _ in range(2)]
        + [pltpu.SemaphoreType.DMA for _ in range(4)]
    ),
)
def _gab(tab, idx, out, iva, ivb, ra, rb, gsa, gsb, wsa, wsb):
    wid = _wid()
    ivs, rs, gs, ws = (iva, ivb), (ra, rb), (gsa, gsb), (wsa, wsb)
    base0 = wid * (_E_NCH * _E_CH)
    wb = [None, None]

    def fire(g):
        s = g & 1
        pltpu.sync_copy(idx.at[pl.ds(base0 + g * _E_CH, _E_CH)], ivs[s])
        return pltpu.async_copy(tab.at[ivs[s]], rs[s], gs[s])

    pending = fire(0)
    for g in range(_E_NCH):
        s = g & 1
        n = 1 - s
        cur = pending
        if g + 1 < _E_NCH:
            if wb[n] is not None:
                wb[n].wait()
            pending = fire(g + 1)
        cur.wait()
        wb[s] = pltpu.async_copy(
            rs[s], out.at[pl.ds(base0 + g * _E_CH, _E_CH)], ws[s])
    for w in wb:
        if w is not None:
            w.wait()


@functools.partial(
    pl.kernel,
    mesh=plsc.VectorSubcoreMesh(**_MESH),
    out_type=[jax.ShapeDtypeStruct((E_PAD, HID), jnp.float32),
              jax.ShapeDtypeStruct((E_PAD, HID), jnp.float32)],
    scratch_types=(
        [pltpu.VMEM((_E_CH,), jnp.int32) for _ in range(4)]
        + [pltpu.VMEM((_E_CH, HID), jnp.float32) for _ in range(4)]
        + [pltpu.SemaphoreType.DMA for _ in range(8)]
    ),
)
def _g12(t1, i1, t2, i2, o1, o2,
         iv1a, iv1b, iv2a, iv2b, r1a, r1b, r2a, r2b,
         g1a, g1b, g2a, g2b, w1a, w1b, w2a, w2b):
    wid = _wid()
    iv1s, iv2s = (iv1a, iv1b), (iv2a, iv2b)
    r1s, r2s = (r1a, r1b), (r2a, r2b)
    g1s, g2s = (g1a, g1b), (g2a, g2b)
    w1s, w2s = (w1a, w1b), (w2a, w2b)
    base0 = wid * (_E_NCH * _E_CH)
    wb = [None, None]

    def fire(g):
        s = g & 1
        sl = pl.ds(base0 + g * _E_CH, _E_CH)
        pltpu.sync_copy(i1.at[sl], iv1s[s])
        pltpu.sync_copy(i2.at[sl], iv2s[s])
        return (pltpu.async_copy(t1.at[iv1s[s]], r1s[s], g1s[s]),
                pltpu.async_copy(t2.at[iv2s[s]], r2s[s], g2s[s]))

    pending = fire(0)
    for g in range(_E_NCH):
        s = g & 1
        n = 1 - s
        cur = pending
        if g + 1 < _E_NCH:
            if wb[n] is not None:
                for w in wb[n]:
                    w.wait()
            pending = fire(g + 1)
        for c in cur:
            c.wait()
        sl = pl.ds(base0 + g * _E_CH, _E_CH)
        wb[s] = (pltpu.async_copy(r1s[s], o1.at[sl], w1s[s]),
                 pltpu.async_copy(r2s[s], o2.at[sl], w2s[s]))
    for pair in wb:
        if pair is not None:
            for w in pair:
                w.wait()


@functools.partial(
    pl.kernel,
    mesh=plsc.VectorSubcoreMesh(**_MESH),
    out_type=jax.ShapeDtypeStruct((N_PAD, HID), jnp.float32),
    scratch_types=(
        [pltpu.VMEM((_A_CH,), jnp.int32) for _ in range(2 * MAX_NB)]
        + [pltpu.VMEM((_A_CH, HID), jnp.float32) for _ in range(2 * MAX_NB)]
        + [pltpu.SemaphoreType.DMA for _ in range(2 * MAX_NB + 2)]
    ),
)
def _sk(msg, cols, s_out,
        i0a, i0b, i1a, i1b, i2a, i2b, i3a, i3b, i4a, i4b, i5a, i5b,
        r0a, r0b, r1a, r1b, r2a, r2b, r3a, r3b, r4a, r4b, r5a, r5b,
        g0a, g0b, g1a, g1b, g2a, g2b, g3a, g3b, g4a, g4b, g5a, g5b,
        wsa, wsb):
    ivs = [(i0a, i0b), (i1a, i1b), (i2a, i2b),
           (i3a, i3b), (i4a, i4b), (i5a, i5b)]
    rs = [(r0a, r0b), (r1a, r1b), (r2a, r2b),
          (r3a, r3b), (r4a, r4b), (r5a, r5b)]
    gsems = [(g0a, g0b), (g1a, g1b), (g2a, g2b),
             (g3a, g3b), (g4a, g4b), (g5a, g5b)]
    ws = (wsa, wsb)
    wid = _wid()
    base0 = wid * (_A_NCH * _A_CH)
    wb = [None, None]

    def fire(g):
        s = g & 1
        base = base0 + g * _A_CH
        out = []
        for j in range(MAX_NB):
            pltpu.sync_copy(cols.at[pl.ds(j * N_PAD + base, _A_CH)], ivs[j][s])
            out.append(pltpu.async_copy(msg.at[ivs[j][s]], rs[j][s], gsems[j][s]))
        return out

    pending = fire(0)
    for g in range(_A_NCH):
        s = g & 1
        n = 1 - s
        cur = pending
        if g + 1 < _A_NCH:
            if wb[n] is not None:
                wb[n].wait()
            pending = fire(g + 1)
        for c in cur:
            c.wait()
        r0, r1, r2, r3, r4, r5 = (rs[j][s] for j in range(MAX_NB))

        def add_row(rr, cc):
            for kk in range(HID // 16):
                sl2 = pl.ds(kk * 16, 16)
                r0[rr, sl2] = (r0[rr, sl2] + r1[rr, sl2] + r2[rr, sl2]
                               + r3[rr, sl2] + r4[rr, sl2] + r5[rr, sl2])
            return cc

        lax.fori_loop(0, _A_CH, add_row, 0)
        wb[s] = pltpu.async_copy(
            r0, s_out.at[pl.ds(base0 + g * _A_CH, _A_CH)], ws[s])
    for w in wb:
        if w is not None:
            w.wait()


_BE = 2048
_EG = E_PAD // _BE


def _proj_body(ag_ref, b_ref, br_ref, wa_ref, wb_ref, ip_ref, m0_ref, c_ref):
    a = ag_ref[:, :ATOM_F]
    g3 = ag_ref[:, ATOM_F:ATOM_F + 16]
    ip = jnp.dot(a, wa_ref[...], preferred_element_type=jnp.float32)
    ip = ip + jnp.dot(b_ref[...], wb_ref[...], preferred_element_type=jnp.float32)
    ip_ref[...] = ip
    m0_ref[...] = jnp.maximum(ip, 0.0)
    eq = (g3 == br_ref[...]).astype(jnp.float32)
    c_ref[...] = jnp.sum(eq, axis=1, keepdims=True)


def _proj(AG, bond_p, brev_col, W_a, W_b):
    return pl.pallas_call(
        _proj_body,
        grid=(_EG,),
        in_specs=[
            pl.BlockSpec((_BE, 2 * ATOM_F), lambda i: (i, 0)),
            pl.BlockSpec((_BE, BOND_F), lambda i: (i, 0)),
            pl.BlockSpec((_BE, 1), lambda i: (i, 0)),
            pl.BlockSpec((ATOM_F, HID), lambda i: (0, 0)),
            pl.BlockSpec((BOND_F, HID), lambda i: (0, 0)),
        ],
        out_specs=[
            pl.BlockSpec((_BE, HID), lambda i: (i, 0)),
            pl.BlockSpec((_BE, HID), lambda i: (i, 0)),
            pl.BlockSpec((_BE, 1), lambda i: (i, 0)),
        ],
        out_shape=[
            jax.ShapeDtypeStruct((E_PAD, HID), jnp.float32),
            jax.ShapeDtypeStruct((E_PAD, HID), jnp.float32),
            jax.ShapeDtypeStruct((E_PAD, 1), jnp.float32),
        ],
    )(AG, bond_p, brev_col, W_a, W_b)


def _depth_body(g1_ref, g2_ref, c_ref, ip_ref, wh_ref, out_ref):
    nei = g1_ref[...] - c_ref[...] * g2_ref[...]
    h = jnp.dot(nei, wh_ref[...], preferred_element_type=jnp.float32)
    out_ref[...] = jnp.maximum(h + ip_ref[...], 0.0)


def _depth(G1, G2, c, ip, W_h2):
    return pl.pallas_call(
        _depth_body,
        grid=(_EG,),
        in_specs=[
            pl.BlockSpec((_BE, HID), lambda i: (i, 0)),
            pl.BlockSpec((_BE, HID), lambda i: (i, 0)),
            pl.BlockSpec((_BE, 1), lambda i: (i, 0)),
            pl.BlockSpec((_BE, HID), lambda i: (i, 0)),
            pl.BlockSpec((HID, HID), lambda i: (0, 0)),
        ],
        out_specs=pl.BlockSpec((_BE, HID), lambda i: (i, 0)),
        out_shape=jax.ShapeDtypeStruct((E_PAD, HID), jnp.float32),
    )(G1, G2, c, ip, W_h2)


_BN = 2000
_NBLK = N_ATOMS // _BN


def _out_body(af_ref, s_ref, bt_ref, woa_ref, woh_ref, bias_ref,
              out_ref, acc_ref, cnt_ref):
    i = pl.program_id(0)

    @pl.when(i == 0)
    def _():
        acc_ref[...] = jnp.zeros_like(acc_ref)
        cnt_ref[...] = jnp.zeros_like(cnt_ref)

    h = jnp.dot(af_ref[...], woa_ref[...], preferred_element_type=jnp.float32)
    h = h + jnp.dot(s_ref[...], woh_ref[...], preferred_element_type=jnp.float32)
    h = jnp.maximum(h + bias_ref[...], 0.0)
    gi = lax.broadcasted_iota(jnp.int32, (NG, _BN), 0)
    bt = bt_ref[...].reshape(1, _BN)
    p = (gi == bt).astype(jnp.float32)
    acc_ref[...] += jnp.dot(p, h, preferred_element_type=jnp.float32)
    cnt_ref[...] += jnp.sum(p, axis=1, keepdims=True)

    @pl.when(i == _NBLK - 1)
    def _():
        out_ref[...] = acc_ref[...] / jnp.maximum(cnt_ref[...], 1.0)


def _outk(atom_feats, s_fin, batch_row, W_oa, W_oh, bias):
    return pl.pallas_call(
        _out_body,
        grid=(_NBLK,),
        in_specs=[
            pl.BlockSpec((_BN, ATOM_F), lambda i: (i, 0)),
            pl.BlockSpec((_BN, HID), lambda i: (i, 0)),
            pl.BlockSpec((1, 1, _BN), lambda i: (i, 0, 0)),
            pl.BlockSpec((ATOM_F, HID), lambda i: (0, 0)),
            pl.BlockSpec((HID, HID), lambda i: (0, 0)),
            pl.BlockSpec((1, HID), lambda i: (0, 0)),
        ],
        out_specs=pl.BlockSpec((NG, HID), lambda i: (0, 0)),
        out_shape=jax.ShapeDtypeStruct((NG, HID), jnp.float32),
        scratch_shapes=[
            pltpu.VMEM((NG, HID), jnp.float32),
            pltpu.VMEM((NG, HID), jnp.float32),
        ],
    )(atom_feats, s_fin, batch_row, W_oa, W_oh, bias)


def kernel(atom_feats, bond_feats, a2b, b2a, b2revb, batch,
           W_i, W_h, W_o_w, W_o_b):
    b2a_p = jnp.pad(b2a, (0, E_PAD - N_BONDS))
    b2revb_p = jnp.pad(b2revb, (0, E_PAD - N_BONDS))
    bond_p = jnp.pad(bond_feats, ((0, E_PAD - N_BONDS), (0, 0)))
    a2b_rows = jnp.pad(a2b, ((0, N_PAD - N_ATOMS), (0, 0)))
    a2b_cols = a2b_rows.T.reshape(-1)
    tab_ag = jnp.concatenate(
        [atom_feats, a2b.astype(jnp.float32),
         jnp.full((N_ATOMS, ATOM_F - MAX_NB), -1.0, jnp.float32)], axis=1)
    brev_col = b2revb_p.astype(jnp.float32).reshape(E_PAD, 1)
    batch_row = batch.reshape(_NBLK, 1, _BN)
    W_a = W_i[:, :ATOM_F].T
    W_b = W_i[:, ATOM_F:].T
    W_h2 = W_h.T
    W_oa = W_o_w[:, :ATOM_F].T
    W_oh = W_o_w[:, ATOM_F:].T
    bias = W_o_b.reshape(1, HID)

    AG = _gab(tab_ag, b2a_p)
    ip, msg, c = _proj(AG, bond_p, brev_col, W_a, W_b)
    for _ in range(DEPTH - 1):
        s = _sk(msg, a2b_cols)
        G1, G2 = _g12(s, b2a_p, msg, b2revb_p)
        msg = _depth(G1, G2, c, ip, W_h2)
    s = _sk(msg, a2b_cols)
    return _outk(atom_feats, s, batch_row, W_oa, W_oh, bias)
